# baseline (speedup 1.0000x reference)
import functools

import jax
import jax.numpy as jnp
import numpy as np
from jax import lax
from jax.experimental import pallas as pl
from jax.experimental.pallas import tpu as pltpu
from jax.sharding import Mesh, PartitionSpec as P

_BF = jnp.bfloat16
_F32 = jnp.float32


def _xt_body(x_ref, o_ref):
    o_ref[...] = x_ref[...].astype(_BF).T


def _xt_call(x):
    n_tok, d = x.shape
    return pl.pallas_call(
        _xt_body,
        out_shape=jax.ShapeDtypeStruct((d, n_tok), _BF),
    )(x)


def _head_body(xt_ref, wt_ref, b_ref, o_ref):
    o_ref[...] = (
        jnp.dot(wt_ref[...].astype(_BF), xt_ref[...], preferred_element_type=_F32)
        + b_ref[...]
    )


def _tail_body(xt_ref, p_ref, w_ref, b_ref, o_ref, h_ref, *, w_transposed):
    @pl.when(pl.program_id(0) == 0)
    def _():
        h_ref[...] = lax.dot_general(
            p_ref[...].astype(_BF),
            xt_ref[...],
            (((0,), (0,)), ((), ())),
            preferred_element_type=_F32,
        ).astype(_BF)

    if w_transposed:
        acc = jnp.dot(w_ref[...].astype(_BF), h_ref[...], preferred_element_type=_F32)
    else:
        acc = lax.dot_general(
            w_ref[...].astype(_BF),
            h_ref[...],
            (((0,), (0,)), ((), ())),
            preferred_element_type=_F32,
        )
    o_ref[...] = acc + b_ref[...]


def _head_call(xt, wt, b, bn):
    d, n_tok = xt.shape
    n_out = wt.shape[0]
    return pl.pallas_call(
        _head_body,
        grid=(pl.cdiv(n_out, bn),),
        in_specs=[
            pl.BlockSpec((d, n_tok), lambda j: (0, 0)),
            pl.BlockSpec((bn, d), lambda j: (j, 0)),
            pl.BlockSpec((bn, 1), lambda j: (j, 0)),
        ],
        out_specs=pl.BlockSpec((bn, n_tok), lambda j: (j, 0)),
        out_shape=jax.ShapeDtypeStruct((n_out, n_tok), _F32),
    )(xt, wt, b)


def _tail_call(xt, p, w, b, bn, w_transposed):
    d, n_tok = xt.shape
    k = p.shape[1]
    n_out = w.shape[0] if w_transposed else w.shape[1]
    if w_transposed:
        w_spec = pl.BlockSpec((bn, k), lambda j: (j, 0))
    else:
        w_spec = pl.BlockSpec((k, bn), lambda j: (0, j))
    return pl.pallas_call(
        functools.partial(_tail_body, w_transposed=w_transposed),
        grid=(pl.cdiv(n_out, bn),),
        in_specs=[
            pl.BlockSpec((d, n_tok), lambda j: (0, 0)),
            pl.BlockSpec((d, k), lambda j: (0, 0)),
            w_spec,
            pl.BlockSpec((bn, 1), lambda j: (j, 0)),
        ],
        out_specs=pl.BlockSpec((bn, n_tok), lambda j: (j, 0)),
        out_shape=jax.ShapeDtypeStruct((n_out, n_tok), _F32),
        scratch_shapes=[pltpu.VMEM((k, n_tok), _BF)],
    )(xt, p, w, b)


def _compute(x, W0t, b0c, P1, W1t, b1c, P2, W2, b2c):
    xt = _xt_call(x)
    lh = _head_call(xt, W0t, b0c, bn=512)
    lc1 = _tail_call(xt, P1, W1t, b1c, bn=1000, w_transposed=True)
    lc2 = _tail_call(xt, P2, W2, b2c, bn=2048, w_transposed=False)
    return lh, lc1, lc2


def kernel(x, W0, b0, P1, W1, b1, P2, W2, b2):
    args = (
        x,
        W0.T,
        b0.reshape(-1, 1),
        P1,
        W1.T,
        b1.reshape(-1, 1),
        P2,
        W2,
        b2.reshape(-1, 1),
    )
    devs = jax.devices()
    if len(devs) >= 2:
        mesh = Mesh(np.array(devs[:2]), ("v",))
        fn = jax.shard_map(
            _compute,
            mesh=mesh,
            in_specs=(
                P(),
                P("v", None),
                P("v", None),
                P(),
                P("v", None),
                P("v", None),
                P(),
                P(None, "v"),
                P("v", None),
            ),
            out_specs=(P("v", None), P("v", None), P("v", None)),
            check_vma=False,
        )
        lh, lc1, lc2 = fn(*args)
    else:
        lh, lc1, lc2 = _compute(*args)
    return (lh.T, lc1.T, lc2.T)


# speedup vs baseline: 1.6649x; 1.6649x over previous
import functools

import jax
import jax.numpy as jnp
from jax import lax
from jax.experimental import pallas as pl
from jax.experimental.pallas import tpu as pltpu

_BF = jnp.bfloat16
_F32 = jnp.float32


def _xt_body(x_ref, o_ref):
    o_ref[...] = x_ref[...].astype(_BF).T


def _xt_call(x):
    n_tok, d = x.shape
    return pl.pallas_call(
        _xt_body,
        out_shape=jax.ShapeDtypeStruct((d, n_tok), _BF),
    )(x)


def _head_body(xt_ref, wt_ref, b_ref, o_ref):
    acc = jnp.dot(
        wt_ref[...].astype(_BF), xt_ref[...], preferred_element_type=_F32
    )
    o_ref[...] = acc + b_ref[...][:, None]


def _tail_body(xt_ref, p_ref, w_ref, b_ref, o_ref, h_ref, *, w_transposed, p_transposed):
    @pl.when(pl.program_id(0) == 0)
    def _():
        if p_transposed:
            h = jnp.dot(
                p_ref[...].astype(_BF), xt_ref[...], preferred_element_type=_F32
            )
        else:
            h = lax.dot_general(
                p_ref[...].astype(_BF),
                xt_ref[...],
                (((0,), (0,)), ((), ())),
                preferred_element_type=_F32,
            )
        h_ref[...] = h.astype(_BF)

    if w_transposed:
        acc = jnp.dot(w_ref[...].astype(_BF), h_ref[...], preferred_element_type=_F32)
    else:
        acc = lax.dot_general(
            w_ref[...].astype(_BF),
            h_ref[...],
            (((0,), (0,)), ((), ())),
            preferred_element_type=_F32,
        )
    o_ref[...] = acc + b_ref[...][:, None]


def _head_call(xt, wt, b, bn):
    d, n_tok = xt.shape
    n_out = wt.shape[0]
    return pl.pallas_call(
        _head_body,
        grid=(pl.cdiv(n_out, bn),),
        in_specs=[
            pl.BlockSpec((d, n_tok), lambda j: (0, 0)),
            pl.BlockSpec((bn, d), lambda j: (j, 0)),
            pl.BlockSpec((bn,), lambda j: (j,)),
        ],
        out_specs=pl.BlockSpec((bn, n_tok), lambda j: (j, 0)),
        out_shape=jax.ShapeDtypeStruct((n_out, n_tok), _F32),
    )(xt, wt, b)


def _tail_call(xt, p, w, b, bn, w_transposed, p_transposed=False):
    d, n_tok = xt.shape
    k = p.shape[0] if p_transposed else p.shape[1]
    n_out = w.shape[0] if w_transposed else w.shape[1]
    if w_transposed:
        w_spec = pl.BlockSpec((bn, k), lambda j: (j, 0))
    else:
        w_spec = pl.BlockSpec((k, bn), lambda j: (0, j))
    p_shape = (k, d) if p_transposed else (d, k)
    return pl.pallas_call(
        functools.partial(
            _tail_body, w_transposed=w_transposed, p_transposed=p_transposed
        ),
        grid=(pl.cdiv(n_out, bn),),
        in_specs=[
            pl.BlockSpec((d, n_tok), lambda j: (0, 0)),
            pl.BlockSpec(p_shape, lambda j: (0, 0)),
            w_spec,
            pl.BlockSpec((bn,), lambda j: (j,)),
        ],
        out_specs=pl.BlockSpec((bn, n_tok), lambda j: (j, 0)),
        out_shape=jax.ShapeDtypeStruct((n_out, n_tok), _F32),
        scratch_shapes=[pltpu.VMEM((k, n_tok), _BF)],
    )(xt, p, w, b)


def kernel(x, W0, b0, P1, W1, b1, P2, W2, b2):
    xt = _xt_call(x)
    lh = _head_call(xt, W0.T, b0, bn=512)
    lc1 = _tail_call(xt, P1, W1.T, b1, bn=1024, w_transposed=True)
    lc2 = _tail_call(
        xt, P2.T, W2, b2, bn=2048, w_transposed=False, p_transposed=True
    )
    return (lh.T, lc1.T, lc2.T)
